# baseline (speedup 1.0000x reference)
import dataclasses
import functools

import jax
import jax.numpy as jnp
from jax.experimental import pallas as pl
from jax.experimental.pallas import tpu as pltpu
from jax.experimental.pallas import tpu_sc as plsc

_NUM_ACT = 96
_SIZE = 51
_RANGE = 4.0
_GRID = 2.0 * _RANGE / (_SIZE - 1)
_INV_GRID = (_SIZE - 1) / (2.0 * _RANGE)
_SHIFT = float(_SIZE // 2)
_TMAX = float(_SIZE - 2)
_HALF_GRID = _GRID / 2.0

_RBLK = 64
_LANES = 16


def _spline_sc(x3, tab0, tab1, sg):
    nrow, h, w = x3.shape
    nch = sg.shape[0]
    tab_len = tab0.shape[0]
    mesh = plsc.VectorSubcoreMesh(
        core_axis_name="core", subcore_axis_name="subcore"
    )

    cp = pltpu.CompilerParams(use_tc_tiling_on_sc=True)
    if "needs_layout_passes" in pltpu.CompilerParams.__dataclass_fields__:
        cp = dataclasses.replace(cp, needs_layout_passes=False)

    @functools.partial(
        pl.kernel,
        mesh=mesh,
        out_type=jax.ShapeDtypeStruct(x3.shape, jnp.float32),
        scratch_types=[
            pltpu.VMEM((tab_len,), jnp.float32),
            pltpu.VMEM((tab_len,), jnp.float32),
            pltpu.VMEM((nch,), jnp.float32),
        ],
        compiler_params=cp,
    )
    def run(x_hbm, tab0_hbm, tab1_hbm, sg_hbm, out_hbm, tab0_v, tab1_v, sg_v):
        pltpu.sync_copy(tab0_hbm, tab0_v)
        pltpu.sync_copy(tab1_hbm, tab1_v)
        pltpu.sync_copy(sg_hbm, sg_v)

        def body(idxs, x_vmem, out_vmem):
            ch = jax.lax.rem(idxs[0], nch)
            chv = jnp.full((_LANES,), ch, jnp.int32)
            sgv = plsc.load_gather(sg_v, [chv])
            basev = chv * _SIZE

            @plsc.parallel_loop(0, _RBLK // 8, 1)
            def _(rt):
                @plsc.parallel_loop(0, w, _LANES, unroll=1)
                def _(c0):
                    for rr in range(8):
                        r = rt * 8 + rr
                        v = x_vmem[0, r, pl.ds(c0, _LANES)]
                        tt = v * sgv + _SHIFT
                        tc = jnp.minimum(jnp.maximum(tt, 0.0), _TMAX)
                        fi = tc.astype(jnp.int32)
                        frac = tt - fi.astype(jnp.float32)
                        idx = basev + fi
                        g0 = plsc.load_gather(tab0_v, [idx])
                        d = plsc.load_gather(tab1_v, [idx])
                        out_vmem[0, r, pl.ds(c0, _LANES)] = g0 + frac * d

        pltpu.emit_pipeline(
            body,
            grid=(nrow, h // _RBLK),
            in_specs=[pl.BlockSpec((1, _RBLK, w), lambda i, j: (i, j, 0))],
            out_specs=[pl.BlockSpec((1, _RBLK, w), lambda i, j: (i, j, 0))],
            core_axis_name=("core", "subcore"),
            dimension_semantics=(pltpu.PARALLEL, pltpu.PARALLEL),
            _explicit_indices=True,
        )(x_hbm, out_hbm)

    return run(x3, tab0, tab1, sg)


def kernel(x, coefficients_vect, scaling_coeffs_vect):
    b, c, h, w = x.shape
    x3 = x.reshape(b * c, h, w)

    s = scaling_coeffs_vect.reshape(c)
    tabf = (
        (coefficients_vect.reshape(c, _SIZE) - jnp.float32(_HALF_GRID))
        / s[:, None]
    ).reshape(-1)
    tab1f = jnp.concatenate([tabf[1:] - tabf[:-1], jnp.zeros((1,), jnp.float32)])
    sg = s * jnp.float32(_INV_GRID)

    out3 = _spline_sc(x3, tabf, tab1f, sg)
    return out3.reshape(x.shape)


# speedup vs baseline: 1.0310x; 1.0310x over previous
import dataclasses
import functools

import jax
import jax.numpy as jnp
from jax.experimental import pallas as pl
from jax.experimental.pallas import tpu as pltpu
from jax.experimental.pallas import tpu_sc as plsc

_NUM_ACT = 96
_SIZE = 51
_RANGE = 4.0
_GRID = 2.0 * _RANGE / (_SIZE - 1)
_INV_GRID = (_SIZE - 1) / (2.0 * _RANGE)
_SHIFT = float(_SIZE // 2)
_TMAX = float(_SIZE - 2)
_HALF_GRID = _GRID / 2.0

_RBLK = 64
_LANES = 16


def _spline_sc(x3, tab0, tab1, sg):
    nrow, h, w = x3.shape
    nch = sg.shape[0]
    tab_len = tab0.shape[0]
    mesh = plsc.VectorSubcoreMesh(
        core_axis_name="core", subcore_axis_name="subcore"
    )

    cp = pltpu.CompilerParams(use_tc_tiling_on_sc=True)
    if "needs_layout_passes" in pltpu.CompilerParams.__dataclass_fields__:
        cp = dataclasses.replace(cp, needs_layout_passes=False)

    @functools.partial(
        pl.kernel,
        mesh=mesh,
        out_type=jax.ShapeDtypeStruct(x3.shape, jnp.float32),
        scratch_types=[
            pltpu.VMEM((tab_len,), jnp.float32),
            pltpu.VMEM((tab_len,), jnp.float32),
            pltpu.VMEM((nch,), jnp.float32),
        ],
        compiler_params=cp,
    )
    def run(x_hbm, tab0_hbm, tab1_hbm, sg_hbm, out_hbm, tab0_v, tab1_v, sg_v):
        pltpu.sync_copy(tab0_hbm, tab0_v)
        pltpu.sync_copy(tab1_hbm, tab1_v)
        pltpu.sync_copy(sg_hbm, sg_v)

        def body(idxs, x_vmem, out_vmem):
            ch = jax.lax.rem(idxs[0], nch)
            chv = jnp.full((_LANES,), ch, jnp.int32)
            sgv = plsc.load_gather(sg_v, [chv])
            basev = chv * _SIZE

            @plsc.parallel_loop(0, _RBLK // 8, 1)
            def _(rt):
                @plsc.parallel_loop(0, w, _LANES, unroll=2)
                def _(c0):
                    for rr in range(8):
                        r = rt * 8 + rr
                        v = x_vmem[0, r, pl.ds(c0, _LANES)]
                        tt = v * sgv + _SHIFT
                        tc = jnp.minimum(jnp.maximum(tt, 0.0), _TMAX)
                        fi = tc.astype(jnp.int32)
                        frac = tt - fi.astype(jnp.float32)
                        idx = basev + fi
                        g0 = plsc.load_gather(tab0_v, [idx])
                        d = plsc.load_gather(tab1_v, [idx])
                        out_vmem[0, r, pl.ds(c0, _LANES)] = g0 + frac * d

        pltpu.emit_pipeline(
            body,
            grid=(nrow, h // _RBLK),
            in_specs=[pl.BlockSpec((1, _RBLK, w), lambda i, j: (i, j, 0))],
            out_specs=[pl.BlockSpec((1, _RBLK, w), lambda i, j: (i, j, 0))],
            core_axis_name=("core", "subcore"),
            dimension_semantics=(pltpu.PARALLEL, pltpu.PARALLEL),
            _explicit_indices=True,
        )(x_hbm, out_hbm)

    return run(x3, tab0, tab1, sg)


def kernel(x, coefficients_vect, scaling_coeffs_vect):
    b, c, h, w = x.shape
    x3 = x.reshape(b * c, h, w)

    s = scaling_coeffs_vect.reshape(c)
    tabf = (
        (coefficients_vect.reshape(c, _SIZE) - jnp.float32(_HALF_GRID))
        / s[:, None]
    ).reshape(-1)
    tab1f = jnp.concatenate([tabf[1:] - tabf[:-1], jnp.zeros((1,), jnp.float32)])
    sg = s * jnp.float32(_INV_GRID)

    out3 = _spline_sc(x3, tabf, tab1f, sg)
    return out3.reshape(x.shape)
